# baseline (speedup 1.0000x reference)
import functools

import jax
import jax.numpy as jnp
from jax import lax
from jax.experimental import pallas as pl
from jax.experimental.pallas import tpu as pltpu
from jax.experimental.pallas import tpu_sc as plsc

F32 = jnp.float32
I32 = jnp.int32
K = 80
BLK = 25
NSUB = 16
NCORE = 2
BN = 400


def _i16(v):
    return jnp.full((16,), v, dtype=I32)


def _mm_head_body(x_ref, w_ref, asrc_ref, adst_ref, *out_refs):
    x = x_ref[...]
    h = jnp.dot(x, w_ref[...], preferred_element_type=F32)
    if len(out_refs) == 5:
        hlo_ref, hhi_ref, es_ref, ed_ref, m_ref = out_refs
        hlo_ref[...] = h[:, :128]
        hhi_ref[...] = h[:, 128:]
    else:
        h_ref, es_ref, ed_ref, m_ref = out_refs
        h_ref[...] = h
    es = jnp.dot(h, asrc_ref[...], preferred_element_type=F32)
    ed = jnp.dot(h, adst_ref[...], preferred_element_type=F32)
    es_ref[...] = es
    ed_ref[...] = ed
    bm = jnp.max(es)
    i = pl.program_id(0)

    @pl.when(i == 0)
    def _():
        m_ref[...] = jnp.full((1, 128), bm, F32)

    @pl.when(i > 0)
    def _():
        m_ref[...] = jnp.maximum(m_ref[...], bm)


def _mm_first(x, W, a_src, a_dst):
    N, in_c = x.shape
    out_c = W.shape[1]
    nb = N // BN
    split = out_c == 256
    if split:
        outs = [jax.ShapeDtypeStruct((N, 128), F32), jax.ShapeDtypeStruct((N, 128), F32)]
        out_specs = [pl.BlockSpec((BN, 128), lambda i: (i, 0)),
                     pl.BlockSpec((BN, 128), lambda i: (i, 0))]
    else:
        outs = [jax.ShapeDtypeStruct((N, out_c), F32)]
        out_specs = [pl.BlockSpec((BN, out_c), lambda i: (i, 0))]
    outs += [jax.ShapeDtypeStruct((N, 1), F32), jax.ShapeDtypeStruct((N, 1), F32),
             jax.ShapeDtypeStruct((1, 128), F32)]
    out_specs += [pl.BlockSpec((BN, 1), lambda i: (i, 0)),
                  pl.BlockSpec((BN, 1), lambda i: (i, 0)),
                  pl.BlockSpec((1, 128), lambda i: (0, 0))]
    return pl.pallas_call(
        _mm_head_body,
        grid=(nb,),
        in_specs=[
            pl.BlockSpec((BN, in_c), lambda i: (i, 0)),
            pl.BlockSpec((in_c, out_c), lambda i: (0, 0)),
            pl.BlockSpec((out_c, 1), lambda i: (0, 0)),
            pl.BlockSpec((out_c, 1), lambda i: (0, 0)),
        ],
        out_specs=out_specs,
        out_shape=outs,
    )(x, W, a_src.reshape(out_c, 1), a_dst.reshape(out_c, 1))


def _mm_mid_body(nlo_ref, nhi_ref, s_ref, bprev_ref, w_ref, asrc_ref, adst_ref,
                 *out_refs):
    inv = 1.0 / (s_ref[...] + 1e-16)
    b = bprev_ref[...]
    xlo = jnp.maximum(nlo_ref[...] * inv + b[:, :128], 0.0)
    xhi = jnp.maximum(nhi_ref[...] * inv + b[:, 128:], 0.0)
    x = jnp.concatenate([xlo, xhi], axis=1)
    h = jnp.dot(x, w_ref[...], preferred_element_type=F32)
    if len(out_refs) == 5:
        hlo_ref, hhi_ref, es_ref, ed_ref, m_ref = out_refs
        hlo_ref[...] = h[:, :128]
        hhi_ref[...] = h[:, 128:]
    else:
        h_ref, es_ref, ed_ref, m_ref = out_refs
        h_ref[...] = h
    es = jnp.dot(h, asrc_ref[...], preferred_element_type=F32)
    ed = jnp.dot(h, adst_ref[...], preferred_element_type=F32)
    es_ref[...] = es
    ed_ref[...] = ed
    bm = jnp.max(es)
    i = pl.program_id(0)

    @pl.when(i == 0)
    def _():
        m_ref[...] = jnp.full((1, 128), bm, F32)

    @pl.when(i > 0)
    def _():
        m_ref[...] = jnp.maximum(m_ref[...], bm)


def _mm_mid(nlo, nhi, s, bprev, W, a_src, a_dst):
    N = nlo.shape[0]
    out_c = W.shape[1]
    nb = N // BN
    split = out_c == 256
    if split:
        outs = [jax.ShapeDtypeStruct((N, 128), F32), jax.ShapeDtypeStruct((N, 128), F32)]
        out_specs = [pl.BlockSpec((BN, 128), lambda i: (i, 0)),
                     pl.BlockSpec((BN, 128), lambda i: (i, 0))]
    else:
        outs = [jax.ShapeDtypeStruct((N, out_c), F32)]
        out_specs = [pl.BlockSpec((BN, out_c), lambda i: (i, 0))]
    outs += [jax.ShapeDtypeStruct((N, 1), F32), jax.ShapeDtypeStruct((N, 1), F32),
             jax.ShapeDtypeStruct((1, 128), F32)]
    out_specs += [pl.BlockSpec((BN, 1), lambda i: (i, 0)),
                  pl.BlockSpec((BN, 1), lambda i: (i, 0)),
                  pl.BlockSpec((1, 128), lambda i: (0, 0))]
    return pl.pallas_call(
        _mm_mid_body,
        grid=(nb,),
        in_specs=[
            pl.BlockSpec((BN, 128), lambda i: (i, 0)),
            pl.BlockSpec((BN, 128), lambda i: (i, 0)),
            pl.BlockSpec((BN, 1), lambda i: (i, 0)),
            pl.BlockSpec((1, 256), lambda i: (0, 0)),
            pl.BlockSpec((256, out_c), lambda i: (0, 0)),
            pl.BlockSpec((out_c, 1), lambda i: (0, 0)),
            pl.BlockSpec((out_c, 1), lambda i: (0, 0)),
        ],
        out_specs=out_specs,
        out_shape=outs,
    )(nlo, nhi, s.reshape(N, 1), bprev.reshape(1, 256), W,
      a_src.reshape(out_c, 1), a_dst.reshape(out_c, 1))


def _fin_body(n0_ref, n1_ref, s0_ref, s1_ref, b_ref, o_ref):
    s = s0_ref[...] + s1_ref[...]
    o_ref[...] = (n0_ref[...] + n1_ref[...]) / (s + 1e-16) + b_ref[...]


def _fin(n0, n1, s0, s1, b):
    N, D = n0.shape
    nb = N // BN
    return pl.pallas_call(
        _fin_body,
        grid=(nb,),
        in_specs=[
            pl.BlockSpec((BN, D), lambda i: (i, 0)),
            pl.BlockSpec((BN, D), lambda i: (i, 0)),
            pl.BlockSpec((BN, 1), lambda i: (i, 0)),
            pl.BlockSpec((BN, 1), lambda i: (i, 0)),
            pl.BlockSpec((1, D), lambda i: (0, 0)),
        ],
        out_specs=pl.BlockSpec((BN, D), lambda i: (i, 0)),
        out_shape=jax.ShapeDtypeStruct((N, D), F32),
    )(n0, n1, s0.reshape(N, 1), s1.reshape(N, 1), b.reshape(1, D))


def _sc_edge_kernel(N, E, feature_split):
    EW = E // NSUB if feature_split else E // (NSUB * NCORE)
    NCH = EW // K
    S_W0, S_W1 = 632, N - 15 * 632
    mesh = plsc.VectorSubcoreMesh(core_axis_name="c", subcore_axis_name="s")

    if feature_split:
        out_type = (jax.ShapeDtypeStruct((N, 128), F32),
                    jax.ShapeDtypeStruct((N, 128), F32),
                    jax.ShapeDtypeStruct((N,), F32))
    else:
        out_type = (jax.ShapeDtypeStruct((N, 128), F32),
                    jax.ShapeDtypeStruct((N, 128), F32),
                    jax.ShapeDtypeStruct((N,), F32),
                    jax.ShapeDtypeStruct((N,), F32))

    NBLK = NCH // BLK

    scratch = dict(
        m_v=pltpu.VMEM((128,), F32),
        srcb=pltpu.VMEM((BLK, K), I32),
        dstb=pltpu.VMEM((BLK, K), I32),
        esb=pltpu.VMEM((3, K), F32),
        edb=pltpu.VMEM((3, K), F32),
        wbuf=pltpu.VMEM((3, K), F32),
        stage=pltpu.VMEM((3 * K, 128), F32),
        zbuf=pltpu.VMEM((640,), F32),
        num_acc=pltpu.VMEM_SHARED((N, 128), F32),
        s_acc=pltpu.VMEM_SHARED((N,), F32),
        gsem0=pltpu.SemaphoreType.DMA,
        gsem1=pltpu.SemaphoreType.DMA,
        gsem2=pltpu.SemaphoreType.DMA,
        ssem0=pltpu.SemaphoreType.DMA,
        ssem1=pltpu.SemaphoreType.DMA,
        ssem2=pltpu.SemaphoreType.DMA,
    )

    def body(*refs):
        if feature_split:
            (hlo, hhi, es_h, ed_h, m_h, src2_h, dst2_h,
             nlo_o, nhi_o, s_o,
             m_v, srcb, dstb, esb, edb, wbuf, stage, zbuf,
             num_acc, s_acc, gsem0, gsem1, gsem2,
             ssem0, ssem1, ssem2) = refs
        else:
            (h_t, es_h, ed_h, m_h, src2_h, dst2_h,
             np0_o, np1_o, s0_o, s1_o,
             m_v, srcb, dstb, esb, edb, wbuf, stage, zbuf,
             num_acc, s_acc, gsem0, gsem1, gsem2,
             ssem0, ssem1, ssem2) = refs
        gsems = (gsem0, gsem1, gsem2)
        ssems = (ssem0, ssem1, ssem2)
        c = lax.axis_index("c")
        sid = lax.axis_index("s")

        pltpu.sync_copy(m_h, m_v)
        if feature_split:
            wid = sid
        else:
            wid = c * NSUB + sid

        zv = jnp.zeros((16,), F32)

        def zrow(r, _):
            for j in range(8):
                stage[r, pl.ds(j * 16, 16)] = zv
            return 0

        lax.fori_loop(0, 3 * K, zrow, 0)
        for j in range(40):
            zbuf[pl.ds(j * 16, 16)] = zv

        @pl.when(sid < 15)
        def _():
            pltpu.async_copy(stage.at[pl.ds(0, 240)],
                             num_acc.at[pl.ds(sid * S_W0, 240)], gsem0)
            pltpu.async_copy(stage.at[pl.ds(0, 240)],
                             num_acc.at[pl.ds(sid * S_W0 + 240, 240)], gsem1)
            pltpu.async_copy(stage.at[pl.ds(0, 152)],
                             num_acc.at[pl.ds(sid * S_W0 + 480, 152)], gsem2)
            pltpu.make_async_copy(stage.at[pl.ds(0, 240)],
                                  num_acc.at[pl.ds(0, 240)], gsem0).wait()
            pltpu.make_async_copy(stage.at[pl.ds(0, 240)],
                                  num_acc.at[pl.ds(0, 240)], gsem1).wait()
            pltpu.make_async_copy(stage.at[pl.ds(0, 152)],
                                  num_acc.at[pl.ds(0, 152)], gsem2).wait()

        @pl.when(sid == 15)
        def _():
            pltpu.async_copy(stage.at[pl.ds(0, 240)],
                             num_acc.at[pl.ds(15 * S_W0, 240)], gsem0)
            pltpu.async_copy(stage.at[pl.ds(0, 240)],
                             num_acc.at[pl.ds(15 * S_W0 + 240, 240)], gsem1)
            pltpu.async_copy(stage.at[pl.ds(0, 40)],
                             num_acc.at[pl.ds(15 * S_W0 + 480, 40)], gsem2)
            pltpu.make_async_copy(stage.at[pl.ds(0, 240)],
                                  num_acc.at[pl.ds(0, 240)], gsem0).wait()
            pltpu.make_async_copy(stage.at[pl.ds(0, 240)],
                                  num_acc.at[pl.ds(0, 240)], gsem1).wait()
            pltpu.make_async_copy(stage.at[pl.ds(0, 40)],
                                  num_acc.at[pl.ds(0, 40)], gsem2).wait()

        if feature_split:
            @pl.when(c == 0)
            def _():
                _zero_s_slice(zbuf, s_acc, sid)
        else:
            _zero_s_slice(zbuf, s_acc, sid)
        plsc.subcore_barrier()

        m16 = m_v[pl.ds(0, 16)]

        def start_gather(g, b):
            idxrow = srcb.at[g]
            didxrow = dstb.at[g]
            dst_st = stage.at[pl.ds(b * K, K)]
            sem = gsems[b]
            if feature_split:
                @pl.when(c == 0)
                def _():
                    pltpu.async_copy(hlo.at[idxrow], dst_st, sem)

                @pl.when(c == 1)
                def _():
                    pltpu.async_copy(hhi.at[idxrow], dst_st, sem)
            else:
                pltpu.async_copy(h_t.at[idxrow], dst_st, sem)
            pltpu.async_copy(es_h.at[idxrow], esb.at[b], sem)
            pltpu.async_copy(ed_h.at[didxrow], edb.at[b], sem)

        def wait_gather(b):
            sem = gsems[b]
            table = hlo if feature_split else h_t
            pltpu.make_async_copy(table.at[srcb.at[0]],
                                  stage.at[pl.ds(b * K, K)], sem).wait()
            pltpu.make_async_copy(es_h.at[srcb.at[0]], esb.at[b], sem).wait()
            pltpu.make_async_copy(ed_h.at[dstb.at[0]], edb.at[b], sem).wait()

        def start_scatter(g, b):
            sem = ssems[b]
            if feature_split:
                @pl.when(c == 0)
                def _():
                    pltpu.async_copy(wbuf.at[b], s_acc.at[dstb.at[g]], sem,
                                     add=True)
            else:
                pltpu.async_copy(wbuf.at[b], s_acc.at[dstb.at[g]], sem,
                                 add=True)
            pltpu.async_copy(stage.at[pl.ds(b * K, K)],
                             num_acc.at[dstb.at[g]], sem, add=True)

        def wait_scatter(b):
            sem = ssems[b]
            if feature_split:
                @pl.when(c == 0)
                def _():
                    pltpu.make_async_copy(wbuf.at[b], s_acc.at[dstb.at[0]],
                                          sem).wait()
            else:
                pltpu.make_async_copy(wbuf.at[b], s_acc.at[dstb.at[0]],
                                      sem).wait()
            pltpu.make_async_copy(stage.at[pl.ds(b * K, K)],
                                  num_acc.at[dstb.at[0]], sem).wait()

        def compute_w(b):
            for j in range(K // 16):
                sl = pl.ds(j * 16, 16)
                ess = esb[b, sl]
                edd = edb[b, sl]
                z = ess + edd
                e = jnp.maximum(z, 0.2 * z)
                zm = edd + m16
                sh = jnp.maximum(zm, 0.2 * zm)
                wbuf[b, sl] = jnp.exp(e - sh)

        def scale(b):
            bi = _i16(b)

            def row10(t, _):
                r0 = 10 * t
                for i in range(10):
                    wspl = plsc.load_gather(wbuf, [bi, _i16(r0 + i)])
                    for j in range(8):
                        sl = pl.ds(j * 16, 16)
                        stage[b * K + r0 + i, sl] = stage[b * K + r0 + i, sl] * wspl
                return 0

            lax.fori_loop(0, K // 10, row10, 0)

        def do_chunk(g, b):
            nb = (b + 1) % 3

            @pl.when(g >= 2)
            def _():
                wait_scatter(nb)

            @pl.when(g + 1 < BLK)
            def _():
                start_gather(g + 1, nb)

            wait_gather(b)
            compute_w(b)
            scale(b)
            start_scatter(g, b)

        def block(bl, _):
            pltpu.sync_copy(src2_h.at[wid, bl], srcb)
            pltpu.sync_copy(dst2_h.at[wid, bl], dstb)
            start_gather(0, 0)

            def triple(t, _):
                for j in range(3):
                    do_chunk(3 * t + j, j)
                return 0

            lax.fori_loop(0, (BLK - 1) // 3, triple, 0)
            do_chunk(BLK - 1, (BLK - 1) % 3)
            wait_scatter((BLK - 2) % 3)
            wait_scatter((BLK - 1) % 3)
            return 0

        lax.fori_loop(0, NBLK, block, 0)

        plsc.subcore_barrier()

        if feature_split:
            @pl.when(c == 0)
            def _():
                _copy_rows(num_acc, nlo_o, stage, sid)
                _copy_s_slice(s_acc, s_o, zbuf, sid)

            @pl.when(c == 1)
            def _():
                _copy_rows(num_acc, nhi_o, stage, sid)
        else:
            @pl.when(c == 0)
            def _():
                _copy_rows(num_acc, np0_o, stage, sid)
                _copy_s_slice(s_acc, s0_o, zbuf, sid)

            @pl.when(c == 1)
            def _():
                _copy_rows(num_acc, np1_o, stage, sid)
                _copy_s_slice(s_acc, s1_o, zbuf, sid)

    def _copy_rows(num_acc, out_ref, stage, sid):
        def hop(off, rows):
            pltpu.sync_copy(num_acc.at[pl.ds(off, rows)], stage.at[pl.ds(0, rows)])
            pltpu.sync_copy(stage.at[pl.ds(0, rows)], out_ref.at[pl.ds(off, rows)])

        @pl.when(sid < 15)
        def _():
            for kk in range(7):
                hop(sid * S_W0 + kk * 80, 80)
            hop(sid * S_W0 + 560, 72)

        @pl.when(sid == 15)
        def _():
            for kk in range(6):
                hop(15 * S_W0 + kk * 80, 80)
            hop(15 * S_W0 + 480, 40)

    def _zero_s_slice(zbuf, s_acc, sid):
        @pl.when(sid < 15)
        def _():
            pltpu.sync_copy(zbuf.at[pl.ds(0, S_W0)],
                            s_acc.at[pl.ds(sid * S_W0, S_W0)])

        @pl.when(sid == 15)
        def _():
            pltpu.sync_copy(zbuf.at[pl.ds(0, S_W1)],
                            s_acc.at[pl.ds(15 * S_W0, S_W1)])

    def _copy_s_slice(s_acc, s_o, zbuf, sid):
        @pl.when(sid < 15)
        def _():
            pltpu.sync_copy(s_acc.at[pl.ds(sid * S_W0, S_W0)],
                            zbuf.at[pl.ds(0, S_W0)])
            pltpu.sync_copy(zbuf.at[pl.ds(0, S_W0)],
                            s_o.at[pl.ds(sid * S_W0, S_W0)])

        @pl.when(sid == 15)
        def _():
            pltpu.sync_copy(s_acc.at[pl.ds(15 * S_W0, S_W1)],
                            zbuf.at[pl.ds(0, S_W1)])
            pltpu.sync_copy(zbuf.at[pl.ds(0, S_W1)],
                            s_o.at[pl.ds(15 * S_W0, S_W1)])

    return pl.kernel(body, out_type=out_type, mesh=mesh,
                     scratch_types=list(scratch.values()),
                     compiler_params=pltpu.CompilerParams(
                         needs_layout_passes=False))


def kernel(x, edge_index, training,
           W1, a_src1, a_dst1, b1,
           W2, a_src2, a_dst2, b2,
           W3, a_src3, a_dst3, b3):
    N = x.shape[0]
    E = edge_index.shape[1]
    nb16 = E // (NSUB * K * BLK)
    nb32 = E // (NSUB * NCORE * K * BLK)
    src16 = edge_index[0].reshape(NSUB, nb16, BLK, K)
    dst16 = edge_index[1].reshape(NSUB, nb16, BLK, K)
    src32 = edge_index[0].reshape(NSUB * NCORE, nb32, BLK, K)
    dst32 = edge_index[1].reshape(NSUB * NCORE, nb32, BLK, K)

    sc12 = _sc_edge_kernel(N, E, feature_split=True)
    sc3 = _sc_edge_kernel(N, E, feature_split=False)

    h1lo, h1hi, es1, ed1, m1 = _mm_first(x, W1, a_src1, a_dst1)
    n1lo, n1hi, s1 = sc12(h1lo, h1hi, es1.reshape(N), ed1.reshape(N),
                          m1.reshape(128), src16, dst16)
    h2lo, h2hi, es2, ed2, m2 = _mm_mid(n1lo, n1hi, s1, b1, W2, a_src2, a_dst2)
    n2lo, n2hi, s2 = sc12(h2lo, h2hi, es2.reshape(N), ed2.reshape(N),
                          m2.reshape(128), src16, dst16)
    h3, es3, ed3, m3 = _mm_mid(n2lo, n2hi, s2, b2, W3, a_src3, a_dst3)
    n3a, n3b, s3a, s3b = sc3(h3, es3.reshape(N), ed3.reshape(N),
                             m3.reshape(128), src32, dst32)
    return _fin(n3a, n3b, s3a, s3b, b3)


# speedup vs baseline: 1.7426x; 1.7426x over previous
import functools

import jax
import jax.numpy as jnp
from jax import lax
from jax.experimental import pallas as pl
from jax.experimental.pallas import tpu as pltpu
from jax.experimental.pallas import tpu_sc as plsc

F32 = jnp.float32
I32 = jnp.int32
K = 80
BLK = 25
NSUB = 16
NCORE = 2
BN = 400


def _i16(v):
    return jnp.full((16,), v, dtype=I32)


def _mm_head_body(x_ref, w_ref, asrc_ref, adst_ref, *out_refs):
    x = x_ref[...]
    h = jnp.dot(x, w_ref[...], preferred_element_type=F32)
    if len(out_refs) == 5:
        hlo_ref, hhi_ref, es_ref, ed_ref, m_ref = out_refs
        hlo_ref[...] = h[:, :128]
        hhi_ref[...] = h[:, 128:]
    else:
        h_ref, es_ref, ed_ref, m_ref = out_refs
        h_ref[...] = h
    es = jnp.dot(h, asrc_ref[...], preferred_element_type=F32)
    ed = jnp.dot(h, adst_ref[...], preferred_element_type=F32)
    es_ref[...] = es
    ed_ref[...] = ed
    bm = jnp.max(es)
    i = pl.program_id(0)

    @pl.when(i == 0)
    def _():
        m_ref[...] = jnp.full((1, 128), bm, F32)

    @pl.when(i > 0)
    def _():
        m_ref[...] = jnp.maximum(m_ref[...], bm)


def _mm_first(x, W, a_src, a_dst):
    N, in_c = x.shape
    out_c = W.shape[1]
    nb = N // BN
    split = out_c == 256
    if split:
        outs = [jax.ShapeDtypeStruct((N, 128), F32), jax.ShapeDtypeStruct((N, 128), F32)]
        out_specs = [pl.BlockSpec((BN, 128), lambda i: (i, 0)),
                     pl.BlockSpec((BN, 128), lambda i: (i, 0))]
    else:
        outs = [jax.ShapeDtypeStruct((N, out_c), F32)]
        out_specs = [pl.BlockSpec((BN, out_c), lambda i: (i, 0))]
    outs += [jax.ShapeDtypeStruct((N, 1), F32), jax.ShapeDtypeStruct((N, 1), F32),
             jax.ShapeDtypeStruct((1, 128), F32)]
    out_specs += [pl.BlockSpec((BN, 1), lambda i: (i, 0)),
                  pl.BlockSpec((BN, 1), lambda i: (i, 0)),
                  pl.BlockSpec((1, 128), lambda i: (0, 0))]
    return pl.pallas_call(
        _mm_head_body,
        grid=(nb,),
        in_specs=[
            pl.BlockSpec((BN, in_c), lambda i: (i, 0)),
            pl.BlockSpec((in_c, out_c), lambda i: (0, 0)),
            pl.BlockSpec((out_c, 1), lambda i: (0, 0)),
            pl.BlockSpec((out_c, 1), lambda i: (0, 0)),
        ],
        out_specs=out_specs,
        out_shape=outs,
    )(x, W, a_src.reshape(out_c, 1), a_dst.reshape(out_c, 1))


def _mm_mid_body(nlo_ref, nhi_ref, s_ref, bprev_ref, w_ref, asrc_ref, adst_ref,
                 *out_refs):
    inv = 1.0 / (s_ref[...] + 1e-16)
    b = bprev_ref[...]
    xlo = jnp.maximum(nlo_ref[...] * inv + b[:, :128], 0.0)
    xhi = jnp.maximum(nhi_ref[...] * inv + b[:, 128:], 0.0)
    x = jnp.concatenate([xlo, xhi], axis=1)
    h = jnp.dot(x, w_ref[...], preferred_element_type=F32)
    if len(out_refs) == 5:
        hlo_ref, hhi_ref, es_ref, ed_ref, m_ref = out_refs
        hlo_ref[...] = h[:, :128]
        hhi_ref[...] = h[:, 128:]
    else:
        h_ref, es_ref, ed_ref, m_ref = out_refs
        h_ref[...] = h
    es = jnp.dot(h, asrc_ref[...], preferred_element_type=F32)
    ed = jnp.dot(h, adst_ref[...], preferred_element_type=F32)
    es_ref[...] = es
    ed_ref[...] = ed
    bm = jnp.max(es)
    i = pl.program_id(0)

    @pl.when(i == 0)
    def _():
        m_ref[...] = jnp.full((1, 128), bm, F32)

    @pl.when(i > 0)
    def _():
        m_ref[...] = jnp.maximum(m_ref[...], bm)


def _mm_mid(nlo, nhi, s, bprev, W, a_src, a_dst):
    N = nlo.shape[0]
    out_c = W.shape[1]
    nb = N // BN
    split = out_c == 256
    if split:
        outs = [jax.ShapeDtypeStruct((N, 128), F32), jax.ShapeDtypeStruct((N, 128), F32)]
        out_specs = [pl.BlockSpec((BN, 128), lambda i: (i, 0)),
                     pl.BlockSpec((BN, 128), lambda i: (i, 0))]
    else:
        outs = [jax.ShapeDtypeStruct((N, out_c), F32)]
        out_specs = [pl.BlockSpec((BN, out_c), lambda i: (i, 0))]
    outs += [jax.ShapeDtypeStruct((N, 1), F32), jax.ShapeDtypeStruct((N, 1), F32),
             jax.ShapeDtypeStruct((1, 128), F32)]
    out_specs += [pl.BlockSpec((BN, 1), lambda i: (i, 0)),
                  pl.BlockSpec((BN, 1), lambda i: (i, 0)),
                  pl.BlockSpec((1, 128), lambda i: (0, 0))]
    return pl.pallas_call(
        _mm_mid_body,
        grid=(nb,),
        in_specs=[
            pl.BlockSpec((BN, 128), lambda i: (i, 0)),
            pl.BlockSpec((BN, 128), lambda i: (i, 0)),
            pl.BlockSpec((BN, 1), lambda i: (i, 0)),
            pl.BlockSpec((1, 256), lambda i: (0, 0)),
            pl.BlockSpec((256, out_c), lambda i: (0, 0)),
            pl.BlockSpec((out_c, 1), lambda i: (0, 0)),
            pl.BlockSpec((out_c, 1), lambda i: (0, 0)),
        ],
        out_specs=out_specs,
        out_shape=outs,
    )(nlo, nhi, s.reshape(N, 1), bprev.reshape(1, 256), W,
      a_src.reshape(out_c, 1), a_dst.reshape(out_c, 1))


def _fin_body(n0_ref, n1_ref, s0_ref, s1_ref, b_ref, o_ref):
    s = s0_ref[...] + s1_ref[...]
    o_ref[...] = (n0_ref[...] + n1_ref[...]) / (s + 1e-16) + b_ref[...]


def _fin(n0, n1, s0, s1, b):
    N, D = n0.shape
    nb = N // BN
    return pl.pallas_call(
        _fin_body,
        grid=(nb,),
        in_specs=[
            pl.BlockSpec((BN, D), lambda i: (i, 0)),
            pl.BlockSpec((BN, D), lambda i: (i, 0)),
            pl.BlockSpec((BN, 1), lambda i: (i, 0)),
            pl.BlockSpec((BN, 1), lambda i: (i, 0)),
            pl.BlockSpec((1, D), lambda i: (0, 0)),
        ],
        out_specs=pl.BlockSpec((BN, D), lambda i: (i, 0)),
        out_shape=jax.ShapeDtypeStruct((N, D), F32),
    )(n0, n1, s0.reshape(N, 1), s1.reshape(N, 1), b.reshape(1, D))


def _sc_edge_kernel(N, E, feature_split):
    EW = E // NSUB if feature_split else E // (NSUB * NCORE)
    NCH = EW // K
    S_W0, S_W1 = 632, N - 15 * 632
    mesh = plsc.VectorSubcoreMesh(core_axis_name="c", subcore_axis_name="s")

    if feature_split:
        out_type = (jax.ShapeDtypeStruct((N, 128), F32),
                    jax.ShapeDtypeStruct((N, 128), F32),
                    jax.ShapeDtypeStruct((N,), F32))
    else:
        out_type = (jax.ShapeDtypeStruct((N, 128), F32),
                    jax.ShapeDtypeStruct((N, 128), F32),
                    jax.ShapeDtypeStruct((N,), F32),
                    jax.ShapeDtypeStruct((N,), F32))

    NBLK = NCH // BLK

    scratch = dict(
        m_v=pltpu.VMEM((128,), F32),
        srcb=pltpu.VMEM((BLK, K), I32),
        dstb=pltpu.VMEM((BLK, K), I32),
        esb=pltpu.VMEM((3, K), F32),
        edb=pltpu.VMEM((3, K), F32),
        wbuf=pltpu.VMEM((3, K), F32),
        stage=pltpu.VMEM((3 * K, 128), F32),
        zbuf=pltpu.VMEM((640,), F32),
        num_acc=pltpu.VMEM_SHARED((N, 128), F32),
        s_acc=pltpu.VMEM_SHARED((N,), F32),
        gsem0=pltpu.SemaphoreType.DMA,
        gsem1=pltpu.SemaphoreType.DMA,
        gsem2=pltpu.SemaphoreType.DMA,
        ssem0=pltpu.SemaphoreType.DMA,
        ssem1=pltpu.SemaphoreType.DMA,
        ssem2=pltpu.SemaphoreType.DMA,
    )

    def body(*refs):
        if feature_split:
            (hlo, hhi, es_h, ed_h, m_h, src2_h, dst2_h,
             nlo_o, nhi_o, s_o,
             m_v, srcb, dstb, esb, edb, wbuf, stage, zbuf,
             num_acc, s_acc, gsem0, gsem1, gsem2,
             ssem0, ssem1, ssem2) = refs
        else:
            (h_t, es_h, ed_h, m_h, src2_h, dst2_h,
             np0_o, np1_o, s0_o, s1_o,
             m_v, srcb, dstb, esb, edb, wbuf, stage, zbuf,
             num_acc, s_acc, gsem0, gsem1, gsem2,
             ssem0, ssem1, ssem2) = refs
        gsems = (gsem0, gsem1, gsem2)
        ssems = (ssem0, ssem1, ssem2)
        c = lax.axis_index("c")
        sid = lax.axis_index("s")

        pltpu.sync_copy(m_h, m_v)
        if feature_split:
            wid = sid
        else:
            wid = c * NSUB + sid

        zv = jnp.zeros((16,), F32)

        def zrow(r, _):
            for j in range(8):
                stage[r, pl.ds(j * 16, 16)] = zv
            return 0

        lax.fori_loop(0, 3 * K, zrow, 0)
        for j in range(40):
            zbuf[pl.ds(j * 16, 16)] = zv

        @pl.when(sid < 15)
        def _():
            pltpu.async_copy(stage.at[pl.ds(0, 240)],
                             num_acc.at[pl.ds(sid * S_W0, 240)], gsem0)
            pltpu.async_copy(stage.at[pl.ds(0, 240)],
                             num_acc.at[pl.ds(sid * S_W0 + 240, 240)], gsem1)
            pltpu.async_copy(stage.at[pl.ds(0, 152)],
                             num_acc.at[pl.ds(sid * S_W0 + 480, 152)], gsem2)
            pltpu.make_async_copy(stage.at[pl.ds(0, 240)],
                                  num_acc.at[pl.ds(0, 240)], gsem0).wait()
            pltpu.make_async_copy(stage.at[pl.ds(0, 240)],
                                  num_acc.at[pl.ds(0, 240)], gsem1).wait()
            pltpu.make_async_copy(stage.at[pl.ds(0, 152)],
                                  num_acc.at[pl.ds(0, 152)], gsem2).wait()

        @pl.when(sid == 15)
        def _():
            pltpu.async_copy(stage.at[pl.ds(0, 240)],
                             num_acc.at[pl.ds(15 * S_W0, 240)], gsem0)
            pltpu.async_copy(stage.at[pl.ds(0, 240)],
                             num_acc.at[pl.ds(15 * S_W0 + 240, 240)], gsem1)
            pltpu.async_copy(stage.at[pl.ds(0, 40)],
                             num_acc.at[pl.ds(15 * S_W0 + 480, 40)], gsem2)
            pltpu.make_async_copy(stage.at[pl.ds(0, 240)],
                                  num_acc.at[pl.ds(0, 240)], gsem0).wait()
            pltpu.make_async_copy(stage.at[pl.ds(0, 240)],
                                  num_acc.at[pl.ds(0, 240)], gsem1).wait()
            pltpu.make_async_copy(stage.at[pl.ds(0, 40)],
                                  num_acc.at[pl.ds(0, 40)], gsem2).wait()

        if feature_split:
            @pl.when(c == 0)
            def _():
                _zero_s_slice(zbuf, s_acc, sid)
        else:
            _zero_s_slice(zbuf, s_acc, sid)
        plsc.subcore_barrier()

        m16 = m_v[pl.ds(0, 16)]

        def start_gather(g, b):
            idxrow = srcb.at[g]
            didxrow = dstb.at[g]
            dst_st = stage.at[pl.ds(b * K, K)]
            sem = gsems[b]
            if feature_split:
                @pl.when(c == 0)
                def _():
                    pltpu.async_copy(hlo.at[idxrow], dst_st, sem)

                @pl.when(c == 1)
                def _():
                    pltpu.async_copy(hhi.at[idxrow], dst_st, sem)
            else:
                pltpu.async_copy(h_t.at[idxrow], dst_st, sem)
            pltpu.async_copy(es_h.at[idxrow], esb.at[b], sem)
            pltpu.async_copy(ed_h.at[didxrow], edb.at[b], sem)

        def wait_gather(b):
            sem = gsems[b]
            table = hlo if feature_split else h_t
            pltpu.make_async_copy(table.at[srcb.at[0]],
                                  stage.at[pl.ds(b * K, K)], sem).wait()
            pltpu.make_async_copy(es_h.at[srcb.at[0]], esb.at[b], sem).wait()
            pltpu.make_async_copy(ed_h.at[dstb.at[0]], edb.at[b], sem).wait()

        def start_scatter(g, b):
            sem = ssems[b]
            if feature_split:
                @pl.when(c == 0)
                def _():
                    pltpu.async_copy(wbuf.at[b], s_acc.at[dstb.at[g]], sem,
                                     add=True)
            else:
                pltpu.async_copy(wbuf.at[b], s_acc.at[dstb.at[g]], sem,
                                 add=True)
            pltpu.async_copy(stage.at[pl.ds(b * K, K)],
                             num_acc.at[dstb.at[g]], sem, add=True)

        def wait_scatter(b):
            sem = ssems[b]
            if feature_split:
                @pl.when(c == 0)
                def _():
                    pltpu.make_async_copy(wbuf.at[b], s_acc.at[dstb.at[0]],
                                          sem).wait()
            else:
                pltpu.make_async_copy(wbuf.at[b], s_acc.at[dstb.at[0]],
                                      sem).wait()
            pltpu.make_async_copy(stage.at[pl.ds(b * K, K)],
                                  num_acc.at[dstb.at[0]], sem).wait()

        def compute_w(b):
            for j in range(K // 16):
                sl = pl.ds(j * 16, 16)
                ess = esb[b, sl]
                edd = edb[b, sl]
                z = ess + edd
                e = jnp.maximum(z, 0.2 * z)
                zm = edd + m16
                sh = jnp.maximum(zm, 0.2 * zm)
                wbuf[b, sl] = jnp.exp(e - sh)

        def scale(b):
            bi = _i16(b)

            def row5(t, _):
                r0 = 5 * t
                for i in range(5):
                    wspl = plsc.load_gather(wbuf, [bi, _i16(r0 + i)])
                    for j in range(8):
                        sl = pl.ds(j * 16, 16)
                        stage[b * K + r0 + i, sl] = stage[b * K + r0 + i, sl] * wspl
                return 0

            lax.fori_loop(0, K // 5, row5, 0)

        def do_chunk(g, b):
            nb = (b + 1) % 3

            @pl.when(g >= 2)
            def _():
                wait_scatter(nb)

            @pl.when(g + 1 < BLK)
            def _():
                start_gather(g + 1, nb)

            wait_gather(b)
            compute_w(b)
            scale(b)
            start_scatter(g, b)

        def block(bl, _):
            pltpu.sync_copy(src2_h.at[wid, bl], srcb)
            pltpu.sync_copy(dst2_h.at[wid, bl], dstb)
            start_gather(0, 0)

            def triple(t, _):
                for j in range(3):
                    do_chunk(3 * t + j, j)
                return 0

            lax.fori_loop(0, (BLK - 1) // 3, triple, 0)
            do_chunk(BLK - 1, (BLK - 1) % 3)
            wait_scatter((BLK - 2) % 3)
            wait_scatter((BLK - 1) % 3)
            return 0

        lax.fori_loop(0, NBLK, block, 0)

        plsc.subcore_barrier()

        if feature_split:
            @pl.when(c == 0)
            def _():
                _copy_rows(num_acc, nlo_o, stage, sid)
                _copy_s_slice(s_acc, s_o, zbuf, sid)

            @pl.when(c == 1)
            def _():
                _copy_rows(num_acc, nhi_o, stage, sid)
        else:
            @pl.when(c == 0)
            def _():
                _copy_rows(num_acc, np0_o, stage, sid)
                _copy_s_slice(s_acc, s0_o, zbuf, sid)

            @pl.when(c == 1)
            def _():
                _copy_rows(num_acc, np1_o, stage, sid)
                _copy_s_slice(s_acc, s1_o, zbuf, sid)

    def _copy_rows(num_acc, out_ref, stage, sid):
        def hop(off, rows):
            pltpu.sync_copy(num_acc.at[pl.ds(off, rows)], stage.at[pl.ds(0, rows)])
            pltpu.sync_copy(stage.at[pl.ds(0, rows)], out_ref.at[pl.ds(off, rows)])

        @pl.when(sid < 15)
        def _():
            for kk in range(7):
                hop(sid * S_W0 + kk * 80, 80)
            hop(sid * S_W0 + 560, 72)

        @pl.when(sid == 15)
        def _():
            for kk in range(6):
                hop(15 * S_W0 + kk * 80, 80)
            hop(15 * S_W0 + 480, 40)

    def _zero_s_slice(zbuf, s_acc, sid):
        @pl.when(sid < 15)
        def _():
            pltpu.sync_copy(zbuf.at[pl.ds(0, S_W0)],
                            s_acc.at[pl.ds(sid * S_W0, S_W0)])

        @pl.when(sid == 15)
        def _():
            pltpu.sync_copy(zbuf.at[pl.ds(0, S_W1)],
                            s_acc.at[pl.ds(15 * S_W0, S_W1)])

    def _copy_s_slice(s_acc, s_o, zbuf, sid):
        @pl.when(sid < 15)
        def _():
            pltpu.sync_copy(s_acc.at[pl.ds(sid * S_W0, S_W0)],
                            zbuf.at[pl.ds(0, S_W0)])
            pltpu.sync_copy(zbuf.at[pl.ds(0, S_W0)],
                            s_o.at[pl.ds(sid * S_W0, S_W0)])

        @pl.when(sid == 15)
        def _():
            pltpu.sync_copy(s_acc.at[pl.ds(15 * S_W0, S_W1)],
                            zbuf.at[pl.ds(0, S_W1)])
            pltpu.sync_copy(zbuf.at[pl.ds(0, S_W1)],
                            s_o.at[pl.ds(15 * S_W0, S_W1)])

    return pl.kernel(body, out_type=out_type, mesh=mesh,
                     scratch_types=list(scratch.values()),
                     compiler_params=pltpu.CompilerParams(
                         needs_layout_passes=False))


def kernel(x, edge_index, training,
           W1, a_src1, a_dst1, b1,
           W2, a_src2, a_dst2, b2,
           W3, a_src3, a_dst3, b3):
    N = x.shape[0]
    E = edge_index.shape[1]
    nb16 = E // (NSUB * K * BLK)
    nb32 = E // (NSUB * NCORE * K * BLK)
    src16 = edge_index[0].reshape(NSUB, nb16, BLK, K)
    dst16 = edge_index[1].reshape(NSUB, nb16, BLK, K)
    src32 = edge_index[0].reshape(NSUB * NCORE, nb32, BLK, K)
    dst32 = edge_index[1].reshape(NSUB * NCORE, nb32, BLK, K)

    sc12 = _sc_edge_kernel(N, E, feature_split=True)
    sc3 = _sc_edge_kernel(N, E, feature_split=False)

    h1lo, h1hi, es1, ed1, m1 = _mm_first(x, W1, a_src1, a_dst1)
    n1lo, n1hi, s1 = sc12(h1lo, h1hi, es1.reshape(N), ed1.reshape(N),
                          m1.reshape(128), src16, dst16)
    h2lo, h2hi, es2, ed2, m2 = _mm_mid(n1lo, n1hi, s1, b1, W2, a_src2, a_dst2)
    n2lo, n2hi, s2 = sc12(h2lo, h2hi, es2.reshape(N), ed2.reshape(N),
                          m2.reshape(128), src16, dst16)
    h3, es3, ed3, m3 = _mm_mid(n2lo, n2hi, s2, b2, W3, a_src3, a_dst3)
    n3a, n3b, s3a, s3b = sc3(h3, es3.reshape(N), ed3.reshape(N),
                             m3.reshape(128), src32, dst32)
    return _fin(n3a, n3b, s3a, s3b, b3)


# speedup vs baseline: 1.8067x; 1.0368x over previous
import functools

import jax
import jax.numpy as jnp
from jax import lax
from jax.experimental import pallas as pl
from jax.experimental.pallas import tpu as pltpu
from jax.experimental.pallas import tpu_sc as plsc

F32 = jnp.float32
I32 = jnp.int32
K = 80
BLK = 25
NSUB = 16
NCORE = 2
BN = 400


def _i16(v):
    return jnp.full((16,), v, dtype=I32)


def _mm_head_body(x_ref, w_ref, asrc_ref, adst_ref, *out_refs):
    x = x_ref[...]
    h = jnp.dot(x, w_ref[...], preferred_element_type=F32)
    if len(out_refs) == 5:
        hlo_ref, hhi_ref, es_ref, ed_ref, m_ref = out_refs
        hlo_ref[...] = h[:, :128]
        hhi_ref[...] = h[:, 128:]
    else:
        h_ref, es_ref, ed_ref, m_ref = out_refs
        h_ref[...] = h
    es = jnp.dot(h, asrc_ref[...], preferred_element_type=F32)
    ed = jnp.dot(h, adst_ref[...], preferred_element_type=F32)
    es_ref[...] = es
    ed_ref[...] = ed
    bm = jnp.max(es)
    i = pl.program_id(0)

    @pl.when(i == 0)
    def _():
        m_ref[...] = jnp.full((1, 128), bm, F32)

    @pl.when(i > 0)
    def _():
        m_ref[...] = jnp.maximum(m_ref[...], bm)


def _mm_first(x, W, a_src, a_dst):
    N, in_c = x.shape
    out_c = W.shape[1]
    nb = N // BN
    split = out_c == 256
    if split:
        outs = [jax.ShapeDtypeStruct((N, 128), F32), jax.ShapeDtypeStruct((N, 128), F32)]
        out_specs = [pl.BlockSpec((BN, 128), lambda i: (i, 0)),
                     pl.BlockSpec((BN, 128), lambda i: (i, 0))]
    else:
        outs = [jax.ShapeDtypeStruct((N, out_c), F32)]
        out_specs = [pl.BlockSpec((BN, out_c), lambda i: (i, 0))]
    outs += [jax.ShapeDtypeStruct((N, 1), F32), jax.ShapeDtypeStruct((N, 1), F32),
             jax.ShapeDtypeStruct((1, 128), F32)]
    out_specs += [pl.BlockSpec((BN, 1), lambda i: (i, 0)),
                  pl.BlockSpec((BN, 1), lambda i: (i, 0)),
                  pl.BlockSpec((1, 128), lambda i: (0, 0))]
    return pl.pallas_call(
        _mm_head_body,
        grid=(nb,),
        in_specs=[
            pl.BlockSpec((BN, in_c), lambda i: (i, 0)),
            pl.BlockSpec((in_c, out_c), lambda i: (0, 0)),
            pl.BlockSpec((out_c, 1), lambda i: (0, 0)),
            pl.BlockSpec((out_c, 1), lambda i: (0, 0)),
        ],
        out_specs=out_specs,
        out_shape=outs,
    )(x, W, a_src.reshape(out_c, 1), a_dst.reshape(out_c, 1))


def _mm_mid_body(nlo_ref, nhi_ref, s_ref, bprev_ref, w_ref, asrc_ref, adst_ref,
                 *out_refs):
    inv = 1.0 / (s_ref[...] + 1e-16)
    b = bprev_ref[...]
    xlo = jnp.maximum(nlo_ref[...] * inv + b[:, :128], 0.0)
    xhi = jnp.maximum(nhi_ref[...] * inv + b[:, 128:], 0.0)
    x = jnp.concatenate([xlo, xhi], axis=1)
    h = jnp.dot(x, w_ref[...], preferred_element_type=F32)
    if len(out_refs) == 5:
        hlo_ref, hhi_ref, es_ref, ed_ref, m_ref = out_refs
        hlo_ref[...] = h[:, :128]
        hhi_ref[...] = h[:, 128:]
    else:
        h_ref, es_ref, ed_ref, m_ref = out_refs
        h_ref[...] = h
    es = jnp.dot(h, asrc_ref[...], preferred_element_type=F32)
    ed = jnp.dot(h, adst_ref[...], preferred_element_type=F32)
    es_ref[...] = es
    ed_ref[...] = ed
    bm = jnp.max(es)
    i = pl.program_id(0)

    @pl.when(i == 0)
    def _():
        m_ref[...] = jnp.full((1, 128), bm, F32)

    @pl.when(i > 0)
    def _():
        m_ref[...] = jnp.maximum(m_ref[...], bm)


def _mm_mid(nlo, nhi, s, bprev, W, a_src, a_dst):
    N = nlo.shape[0]
    out_c = W.shape[1]
    nb = N // BN
    split = out_c == 256
    if split:
        outs = [jax.ShapeDtypeStruct((N, 128), F32), jax.ShapeDtypeStruct((N, 128), F32)]
        out_specs = [pl.BlockSpec((BN, 128), lambda i: (i, 0)),
                     pl.BlockSpec((BN, 128), lambda i: (i, 0))]
    else:
        outs = [jax.ShapeDtypeStruct((N, out_c), F32)]
        out_specs = [pl.BlockSpec((BN, out_c), lambda i: (i, 0))]
    outs += [jax.ShapeDtypeStruct((N, 1), F32), jax.ShapeDtypeStruct((N, 1), F32),
             jax.ShapeDtypeStruct((1, 128), F32)]
    out_specs += [pl.BlockSpec((BN, 1), lambda i: (i, 0)),
                  pl.BlockSpec((BN, 1), lambda i: (i, 0)),
                  pl.BlockSpec((1, 128), lambda i: (0, 0))]
    return pl.pallas_call(
        _mm_mid_body,
        grid=(nb,),
        in_specs=[
            pl.BlockSpec((BN, 128), lambda i: (i, 0)),
            pl.BlockSpec((BN, 128), lambda i: (i, 0)),
            pl.BlockSpec((BN, 1), lambda i: (i, 0)),
            pl.BlockSpec((1, 256), lambda i: (0, 0)),
            pl.BlockSpec((256, out_c), lambda i: (0, 0)),
            pl.BlockSpec((out_c, 1), lambda i: (0, 0)),
            pl.BlockSpec((out_c, 1), lambda i: (0, 0)),
        ],
        out_specs=out_specs,
        out_shape=outs,
    )(nlo, nhi, s.reshape(N, 1), bprev.reshape(1, 256), W,
      a_src.reshape(out_c, 1), a_dst.reshape(out_c, 1))


def _fin_body(n0_ref, n1_ref, s0_ref, s1_ref, b_ref, o_ref):
    s = s0_ref[...] + s1_ref[...]
    o_ref[...] = (n0_ref[...] + n1_ref[...]) / (s + 1e-16) + b_ref[...]


def _fin(n0, n1, s0, s1, b):
    N, D = n0.shape
    nb = N // BN
    return pl.pallas_call(
        _fin_body,
        grid=(nb,),
        in_specs=[
            pl.BlockSpec((BN, D), lambda i: (i, 0)),
            pl.BlockSpec((BN, D), lambda i: (i, 0)),
            pl.BlockSpec((BN, 1), lambda i: (i, 0)),
            pl.BlockSpec((BN, 1), lambda i: (i, 0)),
            pl.BlockSpec((1, D), lambda i: (0, 0)),
        ],
        out_specs=pl.BlockSpec((BN, D), lambda i: (i, 0)),
        out_shape=jax.ShapeDtypeStruct((N, D), F32),
    )(n0, n1, s0.reshape(N, 1), s1.reshape(N, 1), b.reshape(1, D))


def _sc_edge_kernel(N, E, feature_split):
    EW = E // NSUB if feature_split else E // (NSUB * NCORE)
    NCH = EW // K
    S_W0, S_W1 = 632, N - 15 * 632
    mesh = plsc.VectorSubcoreMesh(core_axis_name="c", subcore_axis_name="s")

    if feature_split:
        out_type = (jax.ShapeDtypeStruct((N, 128), F32),
                    jax.ShapeDtypeStruct((N, 128), F32),
                    jax.ShapeDtypeStruct((N,), F32))
    else:
        out_type = (jax.ShapeDtypeStruct((N, 128), F32),
                    jax.ShapeDtypeStruct((N, 128), F32),
                    jax.ShapeDtypeStruct((N,), F32),
                    jax.ShapeDtypeStruct((N,), F32))

    NBLK = NCH // BLK

    scratch = dict(
        m_v=pltpu.VMEM((128,), F32),
        srcb=pltpu.VMEM((BLK, K), I32),
        dstb=pltpu.VMEM((BLK, K), I32),
        esb=pltpu.VMEM((3, K), F32),
        edb=pltpu.VMEM((3, K), F32),
        wbuf=pltpu.VMEM((3, K), F32),
        stage=pltpu.VMEM((3 * K, 128), F32),
        zbuf=pltpu.VMEM((640,), F32),
        num_acc=pltpu.VMEM_SHARED((N, 128), F32),
        s_acc=pltpu.VMEM_SHARED((N,), F32),
        gsem0=pltpu.SemaphoreType.DMA,
        gsem1=pltpu.SemaphoreType.DMA,
        gsem2=pltpu.SemaphoreType.DMA,
        ssem0=pltpu.SemaphoreType.DMA,
        ssem1=pltpu.SemaphoreType.DMA,
        ssem2=pltpu.SemaphoreType.DMA,
        esem0=pltpu.SemaphoreType.DMA,
        esem1=pltpu.SemaphoreType.DMA,
        esem2=pltpu.SemaphoreType.DMA,
    )

    def body(*refs):
        if feature_split:
            (hlo, hhi, es_h, ed_h, m_h, src2_h, dst2_h,
             nlo_o, nhi_o, s_o,
             m_v, srcb, dstb, esb, edb, wbuf, stage, zbuf,
             num_acc, s_acc, gsem0, gsem1, gsem2,
             ssem0, ssem1, ssem2, esem0, esem1, esem2) = refs
        else:
            (h_t, es_h, ed_h, m_h, src2_h, dst2_h,
             np0_o, np1_o, s0_o, s1_o,
             m_v, srcb, dstb, esb, edb, wbuf, stage, zbuf,
             num_acc, s_acc, gsem0, gsem1, gsem2,
             ssem0, ssem1, ssem2, esem0, esem1, esem2) = refs
        gsems = (gsem0, gsem1, gsem2)
        ssems = (ssem0, ssem1, ssem2)
        esems = (esem0, esem1, esem2)
        c = lax.axis_index("c")
        sid = lax.axis_index("s")

        pltpu.sync_copy(m_h, m_v)
        if feature_split:
            wid = sid
        else:
            wid = c * NSUB + sid

        zv = jnp.zeros((16,), F32)

        def zrow(r, _):
            for j in range(8):
                stage[r, pl.ds(j * 16, 16)] = zv
            return 0

        lax.fori_loop(0, 3 * K, zrow, 0)
        for j in range(40):
            zbuf[pl.ds(j * 16, 16)] = zv

        @pl.when(sid < 15)
        def _():
            pltpu.async_copy(stage.at[pl.ds(0, 240)],
                             num_acc.at[pl.ds(sid * S_W0, 240)], gsem0)
            pltpu.async_copy(stage.at[pl.ds(0, 240)],
                             num_acc.at[pl.ds(sid * S_W0 + 240, 240)], gsem1)
            pltpu.async_copy(stage.at[pl.ds(0, 152)],
                             num_acc.at[pl.ds(sid * S_W0 + 480, 152)], gsem2)
            pltpu.make_async_copy(stage.at[pl.ds(0, 240)],
                                  num_acc.at[pl.ds(0, 240)], gsem0).wait()
            pltpu.make_async_copy(stage.at[pl.ds(0, 240)],
                                  num_acc.at[pl.ds(0, 240)], gsem1).wait()
            pltpu.make_async_copy(stage.at[pl.ds(0, 152)],
                                  num_acc.at[pl.ds(0, 152)], gsem2).wait()

        @pl.when(sid == 15)
        def _():
            pltpu.async_copy(stage.at[pl.ds(0, 240)],
                             num_acc.at[pl.ds(15 * S_W0, 240)], gsem0)
            pltpu.async_copy(stage.at[pl.ds(0, 240)],
                             num_acc.at[pl.ds(15 * S_W0 + 240, 240)], gsem1)
            pltpu.async_copy(stage.at[pl.ds(0, 40)],
                             num_acc.at[pl.ds(15 * S_W0 + 480, 40)], gsem2)
            pltpu.make_async_copy(stage.at[pl.ds(0, 240)],
                                  num_acc.at[pl.ds(0, 240)], gsem0).wait()
            pltpu.make_async_copy(stage.at[pl.ds(0, 240)],
                                  num_acc.at[pl.ds(0, 240)], gsem1).wait()
            pltpu.make_async_copy(stage.at[pl.ds(0, 40)],
                                  num_acc.at[pl.ds(0, 40)], gsem2).wait()

        if feature_split:
            @pl.when(c == 0)
            def _():
                _zero_s_slice(zbuf, s_acc, sid)
        else:
            _zero_s_slice(zbuf, s_acc, sid)
        plsc.subcore_barrier()

        m16 = m_v[pl.ds(0, 16)]

        def start_gather(g, b):
            idxrow = srcb.at[g]
            didxrow = dstb.at[g]
            dst_st = stage.at[pl.ds(b * K, K)]
            sem = gsems[b]
            if feature_split:
                @pl.when(c == 0)
                def _():
                    pltpu.async_copy(hlo.at[idxrow], dst_st, sem)

                @pl.when(c == 1)
                def _():
                    pltpu.async_copy(hhi.at[idxrow], dst_st, sem)
            else:
                pltpu.async_copy(h_t.at[idxrow], dst_st, sem)
            pltpu.async_copy(es_h.at[idxrow], esb.at[b], esems[b])
            pltpu.async_copy(ed_h.at[didxrow], edb.at[b], esems[b])

        def wait_esed(b):
            pltpu.make_async_copy(es_h.at[srcb.at[0]], esb.at[b], esems[b]).wait()
            pltpu.make_async_copy(ed_h.at[dstb.at[0]], edb.at[b], esems[b]).wait()

        def wait_rows(b):
            table = hlo if feature_split else h_t
            pltpu.make_async_copy(table.at[srcb.at[0]],
                                  stage.at[pl.ds(b * K, K)], gsems[b]).wait()

        def start_scatter(g, b):
            sem = ssems[b]
            if feature_split:
                @pl.when(c == 0)
                def _():
                    pltpu.async_copy(wbuf.at[b], s_acc.at[dstb.at[g]], sem,
                                     add=True)
            else:
                pltpu.async_copy(wbuf.at[b], s_acc.at[dstb.at[g]], sem,
                                 add=True)
            pltpu.async_copy(stage.at[pl.ds(b * K, K)],
                             num_acc.at[dstb.at[g]], sem, add=True)

        def wait_scatter(b):
            sem = ssems[b]
            if feature_split:
                @pl.when(c == 0)
                def _():
                    pltpu.make_async_copy(wbuf.at[b], s_acc.at[dstb.at[0]],
                                          sem).wait()
            else:
                pltpu.make_async_copy(wbuf.at[b], s_acc.at[dstb.at[0]],
                                      sem).wait()
            pltpu.make_async_copy(stage.at[pl.ds(b * K, K)],
                                  num_acc.at[dstb.at[0]], sem).wait()

        def compute_w(b):
            for j in range(K // 16):
                sl = pl.ds(j * 16, 16)
                ess = esb[b, sl]
                edd = edb[b, sl]
                z = ess + edd
                e = jnp.maximum(z, 0.2 * z)
                zm = edd + m16
                sh = jnp.maximum(zm, 0.2 * zm)
                wbuf[b, sl] = jnp.exp(e - sh)

        def scale(b):
            bi = _i16(b)

            def row5(t, _):
                r0 = 5 * t
                for i in range(5):
                    wspl = plsc.load_gather(wbuf, [bi, _i16(r0 + i)])
                    for j in range(8):
                        sl = pl.ds(j * 16, 16)
                        stage[b * K + r0 + i, sl] = stage[b * K + r0 + i, sl] * wspl
                return 0

            lax.fori_loop(0, K // 5, row5, 0)

        def do_chunk(g, b):
            nb = (b + 1) % 3

            @pl.when(g >= 2)
            def _():
                wait_scatter(nb)

            @pl.when(g + 1 < BLK)
            def _():
                start_gather(g + 1, nb)

            wait_esed(b)
            compute_w(b)
            wait_rows(b)
            scale(b)
            start_scatter(g, b)

        def block(bl, _):
            pltpu.sync_copy(src2_h.at[wid, bl], srcb)
            pltpu.sync_copy(dst2_h.at[wid, bl], dstb)
            start_gather(0, 0)

            def triple(t, _):
                for j in range(3):
                    do_chunk(3 * t + j, j)
                return 0

            lax.fori_loop(0, (BLK - 1) // 3, triple, 0)
            do_chunk(BLK - 1, (BLK - 1) % 3)
            wait_scatter((BLK - 2) % 3)
            wait_scatter((BLK - 1) % 3)
            return 0

        lax.fori_loop(0, NBLK, block, 0)

        plsc.subcore_barrier()

        if feature_split:
            @pl.when(c == 0)
            def _():
                _copy_rows(num_acc, nlo_o, stage, sid)
                _copy_s_slice(s_acc, s_o, zbuf, sid)

            @pl.when(c == 1)
            def _():
                _copy_rows(num_acc, nhi_o, stage, sid)
        else:
            @pl.when(c == 0)
            def _():
                _copy_rows(num_acc, np0_o, stage, sid)
                _copy_s_slice(s_acc, s0_o, zbuf, sid)

            @pl.when(c == 1)
            def _():
                _copy_rows(num_acc, np1_o, stage, sid)
                _copy_s_slice(s_acc, s1_o, zbuf, sid)

    def _copy_rows(num_acc, out_ref, stage, sid):
        def hop(off, rows):
            pltpu.sync_copy(num_acc.at[pl.ds(off, rows)], stage.at[pl.ds(0, rows)])
            pltpu.sync_copy(stage.at[pl.ds(0, rows)], out_ref.at[pl.ds(off, rows)])

        @pl.when(sid < 15)
        def _():
            for kk in range(7):
                hop(sid * S_W0 + kk * 80, 80)
            hop(sid * S_W0 + 560, 72)

        @pl.when(sid == 15)
        def _():
            for kk in range(6):
                hop(15 * S_W0 + kk * 80, 80)
            hop(15 * S_W0 + 480, 40)

    def _zero_s_slice(zbuf, s_acc, sid):
        @pl.when(sid < 15)
        def _():
            pltpu.sync_copy(zbuf.at[pl.ds(0, S_W0)],
                            s_acc.at[pl.ds(sid * S_W0, S_W0)])

        @pl.when(sid == 15)
        def _():
            pltpu.sync_copy(zbuf.at[pl.ds(0, S_W1)],
                            s_acc.at[pl.ds(15 * S_W0, S_W1)])

    def _copy_s_slice(s_acc, s_o, zbuf, sid):
        @pl.when(sid < 15)
        def _():
            pltpu.sync_copy(s_acc.at[pl.ds(sid * S_W0, S_W0)],
                            zbuf.at[pl.ds(0, S_W0)])
            pltpu.sync_copy(zbuf.at[pl.ds(0, S_W0)],
                            s_o.at[pl.ds(sid * S_W0, S_W0)])

        @pl.when(sid == 15)
        def _():
            pltpu.sync_copy(s_acc.at[pl.ds(15 * S_W0, S_W1)],
                            zbuf.at[pl.ds(0, S_W1)])
            pltpu.sync_copy(zbuf.at[pl.ds(0, S_W1)],
                            s_o.at[pl.ds(15 * S_W0, S_W1)])

    return pl.kernel(body, out_type=out_type, mesh=mesh,
                     scratch_types=list(scratch.values()),
                     compiler_params=pltpu.CompilerParams(
                         needs_layout_passes=False))


def kernel(x, edge_index, training,
           W1, a_src1, a_dst1, b1,
           W2, a_src2, a_dst2, b2,
           W3, a_src3, a_dst3, b3):
    N = x.shape[0]
    E = edge_index.shape[1]
    nb16 = E // (NSUB * K * BLK)
    nb32 = E // (NSUB * NCORE * K * BLK)
    src16 = edge_index[0].reshape(NSUB, nb16, BLK, K)
    dst16 = edge_index[1].reshape(NSUB, nb16, BLK, K)
    src32 = edge_index[0].reshape(NSUB * NCORE, nb32, BLK, K)
    dst32 = edge_index[1].reshape(NSUB * NCORE, nb32, BLK, K)

    sc12 = _sc_edge_kernel(N, E, feature_split=True)
    sc3 = _sc_edge_kernel(N, E, feature_split=False)

    h1lo, h1hi, es1, ed1, m1 = _mm_first(x, W1, a_src1, a_dst1)
    n1lo, n1hi, s1 = sc12(h1lo, h1hi, es1.reshape(N), ed1.reshape(N),
                          m1.reshape(128), src16, dst16)
    h2lo, h2hi, es2, ed2, m2 = _mm_mid(n1lo, n1hi, s1, b1, W2, a_src2, a_dst2)
    n2lo, n2hi, s2 = sc12(h2lo, h2hi, es2.reshape(N), ed2.reshape(N),
                          m2.reshape(128), src16, dst16)
    h3, es3, ed3, m3 = _mm_mid(n2lo, n2hi, s2, b2, W3, a_src3, a_dst3)
    n3a, n3b, s3a, s3b = sc3(h3, es3.reshape(N), ed3.reshape(N),
                             m3.reshape(128), src32, dst32)
    return _fin(n3a, n3b, s3a, s3b, b3)


# speedup vs baseline: 1.8096x; 1.0016x over previous
import functools

import jax
import jax.numpy as jnp
from jax import lax
from jax.experimental import pallas as pl
from jax.experimental.pallas import tpu as pltpu
from jax.experimental.pallas import tpu_sc as plsc

F32 = jnp.float32
I32 = jnp.int32
K = 80
BLK = 25
NSUB = 16
NCORE = 2
BN = 400


def _i16(v):
    return jnp.full((16,), v, dtype=I32)


def _mm_head_body(x_ref, w_ref, asrc_ref, adst_ref, *out_refs):
    x = x_ref[...]
    h = jnp.dot(x, w_ref[...], preferred_element_type=F32)
    if len(out_refs) == 5:
        hlo_ref, hhi_ref, es_ref, ed_ref, m_ref = out_refs
        hlo_ref[...] = h[:, :128]
        hhi_ref[...] = h[:, 128:]
    else:
        h_ref, es_ref, ed_ref, m_ref = out_refs
        h_ref[...] = h
    es = jnp.dot(h, asrc_ref[...], preferred_element_type=F32)
    ed = jnp.dot(h, adst_ref[...], preferred_element_type=F32)
    es_ref[...] = es
    ed_ref[...] = ed
    bm = jnp.max(es)
    i = pl.program_id(0)

    @pl.when(i == 0)
    def _():
        m_ref[...] = jnp.full((1, 128), bm, F32)

    @pl.when(i > 0)
    def _():
        m_ref[...] = jnp.maximum(m_ref[...], bm)


def _mm_first(x, W, a_src, a_dst):
    N, in_c = x.shape
    out_c = W.shape[1]
    nb = N // BN
    split = out_c == 256
    if split:
        outs = [jax.ShapeDtypeStruct((N, 128), F32), jax.ShapeDtypeStruct((N, 128), F32)]
        out_specs = [pl.BlockSpec((BN, 128), lambda i: (i, 0)),
                     pl.BlockSpec((BN, 128), lambda i: (i, 0))]
    else:
        outs = [jax.ShapeDtypeStruct((N, out_c), F32)]
        out_specs = [pl.BlockSpec((BN, out_c), lambda i: (i, 0))]
    outs += [jax.ShapeDtypeStruct((N, 1), F32), jax.ShapeDtypeStruct((N, 1), F32),
             jax.ShapeDtypeStruct((1, 128), F32)]
    out_specs += [pl.BlockSpec((BN, 1), lambda i: (i, 0)),
                  pl.BlockSpec((BN, 1), lambda i: (i, 0)),
                  pl.BlockSpec((1, 128), lambda i: (0, 0))]
    return pl.pallas_call(
        _mm_head_body,
        grid=(nb,),
        in_specs=[
            pl.BlockSpec((BN, in_c), lambda i: (i, 0)),
            pl.BlockSpec((in_c, out_c), lambda i: (0, 0)),
            pl.BlockSpec((out_c, 1), lambda i: (0, 0)),
            pl.BlockSpec((out_c, 1), lambda i: (0, 0)),
        ],
        out_specs=out_specs,
        out_shape=outs,
    )(x, W, a_src.reshape(out_c, 1), a_dst.reshape(out_c, 1))


def _mm_mid_body(nlo_ref, nhi_ref, s_ref, bprev_ref, w_ref, asrc_ref, adst_ref,
                 *out_refs):
    inv = 1.0 / (s_ref[...] + 1e-16)
    b = bprev_ref[...]
    xlo = jnp.maximum(nlo_ref[...] * inv + b[:, :128], 0.0)
    xhi = jnp.maximum(nhi_ref[...] * inv + b[:, 128:], 0.0)
    x = jnp.concatenate([xlo, xhi], axis=1)
    h = jnp.dot(x, w_ref[...], preferred_element_type=F32)
    if len(out_refs) == 5:
        hlo_ref, hhi_ref, es_ref, ed_ref, m_ref = out_refs
        hlo_ref[...] = h[:, :128]
        hhi_ref[...] = h[:, 128:]
    else:
        h_ref, es_ref, ed_ref, m_ref = out_refs
        h_ref[...] = h
    es = jnp.dot(h, asrc_ref[...], preferred_element_type=F32)
    ed = jnp.dot(h, adst_ref[...], preferred_element_type=F32)
    es_ref[...] = es
    ed_ref[...] = ed
    bm = jnp.max(es)
    i = pl.program_id(0)

    @pl.when(i == 0)
    def _():
        m_ref[...] = jnp.full((1, 128), bm, F32)

    @pl.when(i > 0)
    def _():
        m_ref[...] = jnp.maximum(m_ref[...], bm)


def _mm_mid(nlo, nhi, s, bprev, W, a_src, a_dst):
    N = nlo.shape[0]
    out_c = W.shape[1]
    nb = N // BN
    split = out_c == 256
    if split:
        outs = [jax.ShapeDtypeStruct((N, 128), F32), jax.ShapeDtypeStruct((N, 128), F32)]
        out_specs = [pl.BlockSpec((BN, 128), lambda i: (i, 0)),
                     pl.BlockSpec((BN, 128), lambda i: (i, 0))]
    else:
        outs = [jax.ShapeDtypeStruct((N, out_c), F32)]
        out_specs = [pl.BlockSpec((BN, out_c), lambda i: (i, 0))]
    outs += [jax.ShapeDtypeStruct((N, 1), F32), jax.ShapeDtypeStruct((N, 1), F32),
             jax.ShapeDtypeStruct((1, 128), F32)]
    out_specs += [pl.BlockSpec((BN, 1), lambda i: (i, 0)),
                  pl.BlockSpec((BN, 1), lambda i: (i, 0)),
                  pl.BlockSpec((1, 128), lambda i: (0, 0))]
    return pl.pallas_call(
        _mm_mid_body,
        grid=(nb,),
        in_specs=[
            pl.BlockSpec((BN, 128), lambda i: (i, 0)),
            pl.BlockSpec((BN, 128), lambda i: (i, 0)),
            pl.BlockSpec((BN, 1), lambda i: (i, 0)),
            pl.BlockSpec((1, 256), lambda i: (0, 0)),
            pl.BlockSpec((256, out_c), lambda i: (0, 0)),
            pl.BlockSpec((out_c, 1), lambda i: (0, 0)),
            pl.BlockSpec((out_c, 1), lambda i: (0, 0)),
        ],
        out_specs=out_specs,
        out_shape=outs,
    )(nlo, nhi, s.reshape(N, 1), bprev.reshape(1, 256), W,
      a_src.reshape(out_c, 1), a_dst.reshape(out_c, 1))


def _fin_body(n0_ref, n1_ref, s0_ref, s1_ref, b_ref, o_ref):
    s = s0_ref[...] + s1_ref[...]
    o_ref[...] = (n0_ref[...] + n1_ref[...]) / (s + 1e-16) + b_ref[...]


def _fin(n0, n1, s0, s1, b):
    N, D = n0.shape
    nb = N // BN
    return pl.pallas_call(
        _fin_body,
        grid=(nb,),
        in_specs=[
            pl.BlockSpec((BN, D), lambda i: (i, 0)),
            pl.BlockSpec((BN, D), lambda i: (i, 0)),
            pl.BlockSpec((BN, 1), lambda i: (i, 0)),
            pl.BlockSpec((BN, 1), lambda i: (i, 0)),
            pl.BlockSpec((1, D), lambda i: (0, 0)),
        ],
        out_specs=pl.BlockSpec((BN, D), lambda i: (i, 0)),
        out_shape=jax.ShapeDtypeStruct((N, D), F32),
    )(n0, n1, s0.reshape(N, 1), s1.reshape(N, 1), b.reshape(1, D))


def _sc_edge_kernel(N, E, feature_split):
    EW = E // NSUB if feature_split else E // (NSUB * NCORE)
    NCH = EW // K
    S_W0, S_W1 = 632, N - 15 * 632
    mesh = plsc.VectorSubcoreMesh(core_axis_name="c", subcore_axis_name="s")

    if feature_split:
        out_type = (jax.ShapeDtypeStruct((N, 128), F32),
                    jax.ShapeDtypeStruct((N, 128), F32),
                    jax.ShapeDtypeStruct((N,), F32))
    else:
        out_type = (jax.ShapeDtypeStruct((N, 128), F32),
                    jax.ShapeDtypeStruct((N, 128), F32),
                    jax.ShapeDtypeStruct((N,), F32),
                    jax.ShapeDtypeStruct((N,), F32))

    NBLK = NCH // BLK

    scratch = dict(
        m_v=pltpu.VMEM((128,), F32),
        srcb=pltpu.VMEM((BLK, K), I32),
        dstb=pltpu.VMEM((BLK, K), I32),
        esb=pltpu.VMEM((3, K), F32),
        edb=pltpu.VMEM((3, K), F32),
        wbuf=pltpu.VMEM((3, K), F32),
        stage=pltpu.VMEM((3 * K, 128), F32),
        zbuf=pltpu.VMEM((640,), F32),
        num_acc=pltpu.VMEM_SHARED((N, 128), F32),
        s_acc=pltpu.VMEM_SHARED((N,), F32),
        gsem0=pltpu.SemaphoreType.DMA,
        gsem1=pltpu.SemaphoreType.DMA,
        gsem2=pltpu.SemaphoreType.DMA,
        ssem0=pltpu.SemaphoreType.DMA,
        ssem1=pltpu.SemaphoreType.DMA,
        ssem2=pltpu.SemaphoreType.DMA,
        esem0=pltpu.SemaphoreType.DMA,
        esem1=pltpu.SemaphoreType.DMA,
        esem2=pltpu.SemaphoreType.DMA,
    )

    def body(*refs):
        if feature_split:
            (hlo, hhi, es_h, ed_h, m_h, src2_h, dst2_h,
             nlo_o, nhi_o, s_o,
             m_v, srcb, dstb, esb, edb, wbuf, stage, zbuf,
             num_acc, s_acc, gsem0, gsem1, gsem2,
             ssem0, ssem1, ssem2, esem0, esem1, esem2) = refs
        else:
            (h_t, es_h, ed_h, m_h, src2_h, dst2_h,
             np0_o, np1_o, s0_o, s1_o,
             m_v, srcb, dstb, esb, edb, wbuf, stage, zbuf,
             num_acc, s_acc, gsem0, gsem1, gsem2,
             ssem0, ssem1, ssem2, esem0, esem1, esem2) = refs
        gsems = (gsem0, gsem1, gsem2)
        ssems = (ssem0, ssem1, ssem2)
        esems = (esem0, esem1, esem2)
        c = lax.axis_index("c")
        sid = lax.axis_index("s")

        pltpu.sync_copy(m_h, m_v)
        if feature_split:
            wid = sid
        else:
            wid = c * NSUB + sid

        zv = jnp.zeros((16,), F32)

        def zrow(r, _):
            for j in range(8):
                stage[r, pl.ds(j * 16, 16)] = zv
            return 0

        lax.fori_loop(0, 3 * K, zrow, 0)
        for j in range(40):
            zbuf[pl.ds(j * 16, 16)] = zv

        @pl.when(sid < 15)
        def _():
            pltpu.async_copy(stage.at[pl.ds(0, 240)],
                             num_acc.at[pl.ds(sid * S_W0, 240)], gsem0)
            pltpu.async_copy(stage.at[pl.ds(0, 240)],
                             num_acc.at[pl.ds(sid * S_W0 + 240, 240)], gsem1)
            pltpu.async_copy(stage.at[pl.ds(0, 152)],
                             num_acc.at[pl.ds(sid * S_W0 + 480, 152)], gsem2)
            pltpu.make_async_copy(stage.at[pl.ds(0, 240)],
                                  num_acc.at[pl.ds(0, 240)], gsem0).wait()
            pltpu.make_async_copy(stage.at[pl.ds(0, 240)],
                                  num_acc.at[pl.ds(0, 240)], gsem1).wait()
            pltpu.make_async_copy(stage.at[pl.ds(0, 152)],
                                  num_acc.at[pl.ds(0, 152)], gsem2).wait()

        @pl.when(sid == 15)
        def _():
            pltpu.async_copy(stage.at[pl.ds(0, 240)],
                             num_acc.at[pl.ds(15 * S_W0, 240)], gsem0)
            pltpu.async_copy(stage.at[pl.ds(0, 240)],
                             num_acc.at[pl.ds(15 * S_W0 + 240, 240)], gsem1)
            pltpu.async_copy(stage.at[pl.ds(0, 40)],
                             num_acc.at[pl.ds(15 * S_W0 + 480, 40)], gsem2)
            pltpu.make_async_copy(stage.at[pl.ds(0, 240)],
                                  num_acc.at[pl.ds(0, 240)], gsem0).wait()
            pltpu.make_async_copy(stage.at[pl.ds(0, 240)],
                                  num_acc.at[pl.ds(0, 240)], gsem1).wait()
            pltpu.make_async_copy(stage.at[pl.ds(0, 40)],
                                  num_acc.at[pl.ds(0, 40)], gsem2).wait()

        if feature_split:
            @pl.when(c == 0)
            def _():
                _zero_s_slice(zbuf, s_acc, sid)
        else:
            _zero_s_slice(zbuf, s_acc, sid)
        plsc.subcore_barrier()

        m16 = m_v[pl.ds(0, 16)]

        def start_rows(g, b):
            idxrow = srcb.at[g]
            dst_st = stage.at[pl.ds(b * K, K)]
            sem = gsems[b]
            if feature_split:
                @pl.when(c == 0)
                def _():
                    pltpu.async_copy(hlo.at[idxrow], dst_st, sem)

                @pl.when(c == 1)
                def _():
                    pltpu.async_copy(hhi.at[idxrow], dst_st, sem)
            else:
                pltpu.async_copy(h_t.at[idxrow], dst_st, sem)

        def start_esed(g, b):
            pltpu.async_copy(es_h.at[srcb.at[g]], esb.at[b], esems[b])
            pltpu.async_copy(ed_h.at[dstb.at[g]], edb.at[b], esems[b])

        def start_gather(g, b):
            start_rows(g, b)
            start_esed(g, b)

        def wait_esed(b):
            pltpu.make_async_copy(es_h.at[srcb.at[0]], esb.at[b], esems[b]).wait()
            pltpu.make_async_copy(ed_h.at[dstb.at[0]], edb.at[b], esems[b]).wait()

        def wait_rows(b):
            table = hlo if feature_split else h_t
            pltpu.make_async_copy(table.at[srcb.at[0]],
                                  stage.at[pl.ds(b * K, K)], gsems[b]).wait()

        def start_scatter(g, b):
            sem = ssems[b]
            if feature_split:
                @pl.when(c == 0)
                def _():
                    pltpu.async_copy(wbuf.at[b], s_acc.at[dstb.at[g]], sem,
                                     add=True)
            else:
                pltpu.async_copy(wbuf.at[b], s_acc.at[dstb.at[g]], sem,
                                 add=True)
            pltpu.async_copy(stage.at[pl.ds(b * K, K)],
                             num_acc.at[dstb.at[g]], sem, add=True)

        def wait_scatter(b):
            sem = ssems[b]
            if feature_split:
                @pl.when(c == 0)
                def _():
                    pltpu.make_async_copy(wbuf.at[b], s_acc.at[dstb.at[0]],
                                          sem).wait()
            else:
                pltpu.make_async_copy(wbuf.at[b], s_acc.at[dstb.at[0]],
                                      sem).wait()
            pltpu.make_async_copy(stage.at[pl.ds(b * K, K)],
                                  num_acc.at[dstb.at[0]], sem).wait()

        def compute_w(b):
            for j in range(K // 16):
                sl = pl.ds(j * 16, 16)
                ess = esb[b, sl]
                edd = edb[b, sl]
                z = ess + edd
                e = jnp.maximum(z, 0.2 * z)
                zm = edd + m16
                sh = jnp.maximum(zm, 0.2 * zm)
                wbuf[b, sl] = jnp.exp(e - sh)

        def scale(b):
            bi = _i16(b)

            def row5(t, _):
                r0 = 5 * t
                for i in range(5):
                    wspl = plsc.load_gather(wbuf, [bi, _i16(r0 + i)])
                    for j in range(8):
                        sl = pl.ds(j * 16, 16)
                        stage[b * K + r0 + i, sl] = stage[b * K + r0 + i, sl] * wspl
                return 0

            lax.fori_loop(0, K // 5, row5, 0)

        def do_chunk(g, b):
            nb = (b + 1) % 3

            @pl.when(g + 1 < BLK)
            def _():
                start_esed(g + 1, nb)

            @pl.when(g >= 2)
            def _():
                wait_scatter(nb)

            @pl.when(g + 1 < BLK)
            def _():
                start_rows(g + 1, nb)

            wait_esed(b)
            compute_w(b)
            wait_rows(b)
            scale(b)
            start_scatter(g, b)

        def block(bl, _):
            pltpu.sync_copy(src2_h.at[wid, bl], srcb)
            pltpu.sync_copy(dst2_h.at[wid, bl], dstb)
            start_gather(0, 0)

            def triple(t, _):
                for j in range(3):
                    do_chunk(3 * t + j, j)
                return 0

            lax.fori_loop(0, (BLK - 1) // 3, triple, 0)
            do_chunk(BLK - 1, (BLK - 1) % 3)
            wait_scatter((BLK - 2) % 3)
            wait_scatter((BLK - 1) % 3)
            return 0

        lax.fori_loop(0, NBLK, block, 0)

        plsc.subcore_barrier()

        if feature_split:
            @pl.when(c == 0)
            def _():
                _copy_rows(num_acc, nlo_o, stage, sid)
                _copy_s_slice(s_acc, s_o, zbuf, sid)

            @pl.when(c == 1)
            def _():
                _copy_rows(num_acc, nhi_o, stage, sid)
        else:
            @pl.when(c == 0)
            def _():
                _copy_rows(num_acc, np0_o, stage, sid)
                _copy_s_slice(s_acc, s0_o, zbuf, sid)

            @pl.when(c == 1)
            def _():
                _copy_rows(num_acc, np1_o, stage, sid)
                _copy_s_slice(s_acc, s1_o, zbuf, sid)

    def _copy_rows(num_acc, out_ref, stage, sid):
        def hop(off, rows):
            pltpu.sync_copy(num_acc.at[pl.ds(off, rows)], stage.at[pl.ds(0, rows)])
            pltpu.sync_copy(stage.at[pl.ds(0, rows)], out_ref.at[pl.ds(off, rows)])

        @pl.when(sid < 15)
        def _():
            for kk in range(7):
                hop(sid * S_W0 + kk * 80, 80)
            hop(sid * S_W0 + 560, 72)

        @pl.when(sid == 15)
        def _():
            for kk in range(6):
                hop(15 * S_W0 + kk * 80, 80)
            hop(15 * S_W0 + 480, 40)

    def _zero_s_slice(zbuf, s_acc, sid):
        @pl.when(sid < 15)
        def _():
            pltpu.sync_copy(zbuf.at[pl.ds(0, S_W0)],
                            s_acc.at[pl.ds(sid * S_W0, S_W0)])

        @pl.when(sid == 15)
        def _():
            pltpu.sync_copy(zbuf.at[pl.ds(0, S_W1)],
                            s_acc.at[pl.ds(15 * S_W0, S_W1)])

    def _copy_s_slice(s_acc, s_o, zbuf, sid):
        @pl.when(sid < 15)
        def _():
            pltpu.sync_copy(s_acc.at[pl.ds(sid * S_W0, S_W0)],
                            zbuf.at[pl.ds(0, S_W0)])
            pltpu.sync_copy(zbuf.at[pl.ds(0, S_W0)],
                            s_o.at[pl.ds(sid * S_W0, S_W0)])

        @pl.when(sid == 15)
        def _():
            pltpu.sync_copy(s_acc.at[pl.ds(15 * S_W0, S_W1)],
                            zbuf.at[pl.ds(0, S_W1)])
            pltpu.sync_copy(zbuf.at[pl.ds(0, S_W1)],
                            s_o.at[pl.ds(15 * S_W0, S_W1)])

    return pl.kernel(body, out_type=out_type, mesh=mesh,
                     scratch_types=list(scratch.values()),
                     compiler_params=pltpu.CompilerParams(
                         needs_layout_passes=False))


def kernel(x, edge_index, training,
           W1, a_src1, a_dst1, b1,
           W2, a_src2, a_dst2, b2,
           W3, a_src3, a_dst3, b3):
    N = x.shape[0]
    E = edge_index.shape[1]
    nb16 = E // (NSUB * K * BLK)
    nb32 = E // (NSUB * NCORE * K * BLK)
    src16 = edge_index[0].reshape(NSUB, nb16, BLK, K)
    dst16 = edge_index[1].reshape(NSUB, nb16, BLK, K)
    src32 = edge_index[0].reshape(NSUB * NCORE, nb32, BLK, K)
    dst32 = edge_index[1].reshape(NSUB * NCORE, nb32, BLK, K)

    sc12 = _sc_edge_kernel(N, E, feature_split=True)
    sc3 = _sc_edge_kernel(N, E, feature_split=False)

    h1lo, h1hi, es1, ed1, m1 = _mm_first(x, W1, a_src1, a_dst1)
    n1lo, n1hi, s1 = sc12(h1lo, h1hi, es1.reshape(N), ed1.reshape(N),
                          m1.reshape(128), src16, dst16)
    h2lo, h2hi, es2, ed2, m2 = _mm_mid(n1lo, n1hi, s1, b1, W2, a_src2, a_dst2)
    n2lo, n2hi, s2 = sc12(h2lo, h2hi, es2.reshape(N), ed2.reshape(N),
                          m2.reshape(128), src16, dst16)
    h3, es3, ed3, m3 = _mm_mid(n2lo, n2hi, s2, b2, W3, a_src3, a_dst3)
    n3a, n3b, s3a, s3b = sc3(h3, es3.reshape(N), ed3.reshape(N),
                             m3.reshape(128), src32, dst32)
    return _fin(n3a, n3b, s3a, s3b, b3)
